# Initial kernel scaffold; baseline (speedup 1.0000x reference)
import jax
import jax.numpy as jnp
from jax.experimental import pallas as pl


def kernel(score_map, delta_map, anchors):
    raise NotImplementedError("write your pallas kernel here")


# speedup vs baseline: 2.5802x; 2.5802x over previous
import jax
import jax.numpy as jnp
from jax.experimental import pallas as pl
from jax.experimental.pallas import tpu as pltpu

_B, _H, _W, _K = 4, 48, 48, 9
_N = _H * _W * _K
_MOS = 100
_IOU_T = 0.9
_SCORE_T = 0.9
_PROP_T = 0.5
_MPAD = 128


def _nms_body(score_ref, d_ref, a_ref, out_ref):
    scores = score_ref[...]
    tx = d_ref[0:_B]
    ty = d_ref[_B : 2 * _B]
    tw = d_ref[2 * _B : 3 * _B]
    th = d_ref[3 * _B : 4 * _B]
    a0 = a_ref[0:1]
    a1 = a_ref[1:2]
    a2 = a_ref[2:3]
    a3 = a_ref[3:4]

    xa = (a0 + a1) / 2.0
    ya = (a2 + a3) / 2.0
    wa = a1 - a0
    ha = a3 - a2
    x = tx * wa + xa
    y = ty * ha + ya
    w = jnp.exp(tw) * wa
    h = jnp.exp(th) * ha
    oxmin = jnp.maximum(x - w / 2.0, 0.0)
    oxmax = jnp.minimum(x + w / 2.0, 1.0)
    oymin = jnp.maximum(y - h / 2.0, 0.0)
    oymax = jnp.minimum(y + h / 2.0, 1.0)
    ymin = jnp.minimum(oymax, oymin)
    ymax = jnp.maximum(oymax, oymin)
    xmin = jnp.minimum(oxmin, oxmax)
    xmax = jnp.maximum(oxmin, oxmax)
    area = (ymax - ymin) * (xmax - xmin)

    sc0 = jnp.where(scores > _PROP_T, scores, -jnp.inf)
    iota = jax.lax.broadcasted_iota(jnp.int32, (_B, _N), 1)
    lane = jax.lax.broadcasted_iota(jnp.int32, (_B, _MPAD), 1)

    def step(t, carry):
        sc, o0, o1, o2, o3 = carry
        m = jnp.max(sc, axis=1, keepdims=True)
        big_idx = jnp.where(sc == m, iota, _N)
        idx = jnp.min(big_idx, axis=1, keepdims=True)
        onehot = big_idx == idx
        onef = onehot.astype(jnp.float32)
        b_oymax = jnp.sum(oymax * onef, axis=1, keepdims=True)
        b_oxmin = jnp.sum(oxmin * onef, axis=1, keepdims=True)
        b_oymin = jnp.sum(oymin * onef, axis=1, keepdims=True)
        b_oxmax = jnp.sum(oxmax * onef, axis=1, keepdims=True)
        b_ymin = jnp.minimum(b_oymax, b_oymin)
        b_ymax = jnp.maximum(b_oymax, b_oymin)
        b_xmin = jnp.minimum(b_oxmin, b_oxmax)
        b_xmax = jnp.maximum(b_oxmin, b_oxmax)
        b_area = (b_ymax - b_ymin) * (b_xmax - b_xmin)
        iy1 = jnp.maximum(b_ymin, ymin)
        iy2 = jnp.minimum(b_ymax, ymax)
        ix1 = jnp.maximum(b_xmin, xmin)
        ix2 = jnp.minimum(b_xmax, xmax)
        inter = jnp.clip(iy2 - iy1, 0.0, None) * jnp.clip(ix2 - ix1, 0.0, None)
        iou = inter / (b_area + area - inter + 1e-8)
        sc = jnp.where((iou > _IOU_T) | onehot, -jnp.inf, sc)
        valid = (m > _SCORE_T).astype(jnp.float32)
        sel = lane == t
        o0 = jnp.where(sel, b_oymax * valid, o0)
        o1 = jnp.where(sel, b_oxmin * valid, o1)
        o2 = jnp.where(sel, b_oymin * valid, o2)
        o3 = jnp.where(sel, b_oxmax * valid, o3)
        return sc, o0, o1, o2, o3

    zeros = jnp.zeros((_B, _MPAD), jnp.float32)
    _, o0, o1, o2, o3 = jax.lax.fori_loop(
        0, _MOS, step, (sc0, zeros, zeros, zeros, zeros)
    )
    out_ref[0] = o0
    out_ref[1] = o1
    out_ref[2] = o2
    out_ref[3] = o3


def _run(scores, d_pack, a_pack, interpret=False):
    return pl.pallas_call(
        _nms_body,
        out_shape=jax.ShapeDtypeStruct((4, _B, _MPAD), jnp.float32),
        interpret=interpret,
    )(scores, d_pack, a_pack)


def kernel(score_map, delta_map, anchors, interpret=False):
    scores = score_map.reshape(_B, _N)
    d5 = delta_map.reshape(_B, _H * _W * _K, 4).transpose(2, 0, 1)
    d_pack = d5.reshape(4 * _B, _N)
    a_pack = anchors.reshape(_N, 4).T
    out = _run(scores, d_pack, a_pack, interpret=interpret)
    return out.transpose(1, 2, 0)[:, :_MOS, :]


# speedup vs baseline: 4.5063x; 1.7465x over previous
import jax
import jax.numpy as jnp
from jax.experimental import pallas as pl
from jax.experimental.pallas import tpu as pltpu

_B, _H, _W, _K = 4, 48, 48, 9
_N = _H * _W * _K
_NP = 21504
_R = _NP // 128
_MOS = 100
_IOU_T = 0.9
_SCORE_T = 0.9
_PROP_T = 0.5
_MPAD = 128
_BIG = 2**30


def _nms_body(score_ref, d_ref, a_ref, out_ref):
    scores = score_ref[...]
    tx = d_ref[0]
    ty = d_ref[1]
    tw = d_ref[2]
    th = d_ref[3]
    a0 = a_ref[0]
    a1 = a_ref[1]
    a2 = a_ref[2]
    a3 = a_ref[3]

    xa = (a0 + a1) / 2.0
    ya = (a2 + a3) / 2.0
    wa = a1 - a0
    ha = a3 - a2
    x = tx * wa + xa
    y = ty * ha + ya
    w = jnp.exp(tw) * wa
    h = jnp.exp(th) * ha
    oxmin = jnp.maximum(x - w / 2.0, 0.0)
    oxmax = jnp.minimum(x + w / 2.0, 1.0)
    oymin = jnp.maximum(y - h / 2.0, 0.0)
    oymax = jnp.minimum(y + h / 2.0, 1.0)
    ymin = jnp.minimum(oymax, oymin)
    ymax = jnp.maximum(oymax, oymin)
    xmin = jnp.minimum(oxmin, oxmax)
    xmax = jnp.maximum(oxmin, oxmax)
    area = (ymax - ymin) * (xmax - xmin)

    sc0 = jnp.where(scores > _PROP_T, scores, -jnp.inf)
    sub = jax.lax.broadcasted_iota(jnp.int32, (_B, _R, 128), 1)
    lanei = jax.lax.broadcasted_iota(jnp.int32, (_B, _R, 128), 2)
    iota = sub * 128 + lanei
    lane = jax.lax.broadcasted_iota(jnp.int32, (_B, 1, _MPAD), 2)

    def step(t, carry):
        sc, o0, o1, o2, o3 = carry
        m = jnp.max(sc, axis=(1, 2), keepdims=True)
        big_idx = jnp.where(sc == m, iota, _BIG)
        idx = jnp.min(big_idx, axis=(1, 2), keepdims=True)
        onehot = big_idx == idx
        onef = onehot.astype(jnp.float32)
        b_oymax = jnp.sum(oymax * onef, axis=(1, 2), keepdims=True)
        b_oxmin = jnp.sum(oxmin * onef, axis=(1, 2), keepdims=True)
        b_oymin = jnp.sum(oymin * onef, axis=(1, 2), keepdims=True)
        b_oxmax = jnp.sum(oxmax * onef, axis=(1, 2), keepdims=True)
        b_ymin = jnp.minimum(b_oymax, b_oymin)
        b_ymax = jnp.maximum(b_oymax, b_oymin)
        b_xmin = jnp.minimum(b_oxmin, b_oxmax)
        b_xmax = jnp.maximum(b_oxmin, b_oxmax)
        b_area = (b_ymax - b_ymin) * (b_xmax - b_xmin)
        iy1 = jnp.maximum(b_ymin, ymin)
        iy2 = jnp.minimum(b_ymax, ymax)
        ix1 = jnp.maximum(b_xmin, xmin)
        ix2 = jnp.minimum(b_xmax, xmax)
        inter = jnp.clip(iy2 - iy1, 0.0, None) * jnp.clip(ix2 - ix1, 0.0, None)
        iou = inter / (b_area + area - inter + 1e-8)
        sc = jnp.where((iou > _IOU_T) | onehot, -jnp.inf, sc)
        valid = (m > _SCORE_T).astype(jnp.float32)
        sel = lane == t
        o0 = jnp.where(sel, b_oymax * valid, o0)
        o1 = jnp.where(sel, b_oxmin * valid, o1)
        o2 = jnp.where(sel, b_oymin * valid, o2)
        o3 = jnp.where(sel, b_oxmax * valid, o3)
        return sc, o0, o1, o2, o3

    zeros = jnp.zeros((_B, 1, _MPAD), jnp.float32)
    _, o0, o1, o2, o3 = jax.lax.fori_loop(
        0, _MOS, step, (sc0, zeros, zeros, zeros, zeros)
    )
    out_ref[0] = o0[:, 0, :]
    out_ref[1] = o1[:, 0, :]
    out_ref[2] = o2[:, 0, :]
    out_ref[3] = o3[:, 0, :]


def _repack(x, pad_value):
    xp = jnp.pad(x, ((0, 0), (0, _NP - _N)), constant_values=pad_value)
    return xp.reshape(x.shape[0], _R, 128)


def _run(scores, d_pack, a_pack, interpret=False):
    return pl.pallas_call(
        _nms_body,
        out_shape=jax.ShapeDtypeStruct((4, _B, _MPAD), jnp.float32),
        interpret=interpret,
    )(scores, d_pack, a_pack)


def kernel(score_map, delta_map, anchors, interpret=False):
    scores = _repack(score_map.reshape(_B, _N), -jnp.inf)
    d5 = delta_map.reshape(_B, _N, 4).transpose(2, 0, 1)
    d_pack = _repack(d5.reshape(4 * _B, _N), 0.0).reshape(4, _B, _R, 128)
    a5 = anchors.reshape(_N, 4).T
    a_pack = _repack(a5, 0.0).reshape(4, 1, _R, 128)
    a_pack = jnp.broadcast_to(a_pack, (4, _B, _R, 128))
    out = _run(scores, d_pack, a_pack, interpret=interpret)
    return out.transpose(1, 2, 0)[:, :_MOS, :]
